# Initial kernel scaffold; baseline (speedup 1.0000x reference)
import jax
import jax.numpy as jnp
from jax.experimental import pallas as pl


def kernel(mem, pos_enc, token_scores, xy_level, W, b, Ws, B_fourier, layer_token_indices):
    raise NotImplementedError("write your pallas kernel here")


# speedup vs baseline: 10.6877x; 10.6877x over previous
import functools

import jax
import jax.numpy as jnp
from jax import lax
from jax.experimental import pallas as pl
from jax.experimental.pallas import tpu as pltpu
from jax.experimental.pallas import tpu_sc as plsc

NC = 2
NS = 16
NW = NC * NS
C = 128

_MESH = plsc.VectorSubcoreMesh(core_axis_name="c", subcore_axis_name="s")


def _wid():
    return lax.axis_index("s") * NC + lax.axis_index("c")


def _p_gather(pos_enc, idx_lw, L, Tp, D):
    K = Tp // (NW * C)
    BW = K * C

    @functools.partial(
        pl.kernel,
        out_type=jax.ShapeDtypeStruct((L * Tp, D), jnp.float32),
        mesh=_MESH,
        scratch_types=[
            pltpu.VMEM((K, C), jnp.int32),
            pltpu.VMEM((2, C, D), jnp.float32),
            pltpu.SemaphoreType.DMA,
        ],
    )
    def k(p_hbm, idx_hbm, p_out, idx_v, rows_v, sem):
        w = _wid()
        for l in range(L):
            pltpu.sync_copy(idx_hbm.at[l * NW + w], idx_v)
            base = l * Tp + w * BW
            copies = [None, None]
            copies[0] = pltpu.async_copy(p_hbm.at[idx_v.at[0]], rows_v.at[0], sem)
            for j in range(K):
                nj = j + 1
                if nj < K:
                    copies[nj % 2] = pltpu.async_copy(
                        p_hbm.at[idx_v.at[nj]], rows_v.at[nj % 2], sem)
                copies[j % 2].wait()
                pltpu.sync_copy(rows_v.at[j % 2],
                                p_out.at[pl.ds(base + j * C, C)])

    return k(pos_enc, idx_lw)


def _s_gather(scores_mat, srow_lw, scol_lw, L, Tp):
    K = Tp // (NW * C)
    BW = K * C

    @functools.partial(
        pl.kernel,
        out_type=jax.ShapeDtypeStruct((L * Tp,), jnp.float32),
        mesh=_MESH,
        compiler_params=pltpu.CompilerParams(needs_layout_passes=False),
        scratch_types=[
            pltpu.VMEM((K, C), jnp.int32),
            pltpu.VMEM((K, C), jnp.int32),
            pltpu.VMEM((C, 128), jnp.float32),
            pltpu.VMEM((C,), jnp.float32),
            pltpu.SemaphoreType.DMA,
        ],
    )
    def k(s_hbm, srow_hbm, scol_hbm, s_out, srow_v, scol_v, rows_v, sbuf_v,
          sem):
        w = _wid()
        for l in range(L):
            pltpu.sync_copy(srow_hbm.at[l * NW + w], srow_v)
            pltpu.sync_copy(scol_hbm.at[l * NW + w], scol_v)
            base = l * Tp + w * BW
            for j in range(K):
                pltpu.async_copy(s_hbm.at[srow_v.at[j]], rows_v, sem).wait()
                for g in range(C // 16):
                    rows16 = g * 16 + lax.iota(jnp.int32, 16)
                    cols16 = scol_v[j, pl.ds(g * 16, 16)]
                    sbuf_v[pl.ds(g * 16, 16)] = plsc.load_gather(
                        rows_v, [rows16, cols16])
                pltpu.sync_copy(sbuf_v, s_out.at[pl.ds(base + j * C, C)])

    return k(scores_mat, srow_lw, scol_lw)


def _q_gather(x_ref, idx3, Tp, D):
    K = Tp // (NW * C)
    BW = K * C

    @functools.partial(
        pl.kernel,
        out_type=jax.ShapeDtypeStruct((Tp, D), jnp.float32),
        mesh=_MESH,
        scratch_types=[
            pltpu.VMEM((K, C), jnp.int32),
            pltpu.VMEM((2, C, D), jnp.float32),
            pltpu.SemaphoreType.DMA,
        ],
    )
    def k(x_hbm, idx_hbm, q_out, idx_v, rows_v, sem):
        w = _wid()
        base = w * BW
        pltpu.sync_copy(idx_hbm.at[w], idx_v)
        copies = [None, None]
        copies[0] = pltpu.async_copy(x_hbm.at[idx_v.at[0]], rows_v.at[0], sem)
        for j in range(K):
            nj = j + 1
            if nj < K:
                copies[nj % 2] = pltpu.async_copy(
                    x_hbm.at[idx_v.at[nj]], rows_v.at[nj % 2], sem)
            copies[j % 2].wait()
            pltpu.sync_copy(rows_v.at[j % 2], q_out.at[pl.ds(base + j * C, C)])

    return k(x_ref, idx3)


def _row_scatter(rows, idx3, x_ref, Tp, D):
    K = Tp // (NW * C)
    BW = K * C

    @functools.partial(
        pl.kernel,
        mesh=_MESH,
        scratch_types=[
            pltpu.VMEM((K, C), jnp.int32),
            pltpu.VMEM((2, C, D), jnp.float32),
            pltpu.SemaphoreType.DMA,
            pltpu.SemaphoreType.DMA,
        ],
    )
    def k(rows_hbm, idx_hbm, x_hbm, idx_v, rows_v, isem, osem):
        w = _wid()
        base = w * BW
        pltpu.sync_copy(idx_hbm.at[w], idx_v)
        prev = None
        for j in range(K):
            pltpu.sync_copy(rows_hbm.at[pl.ds(base + j * C, C)], rows_v.at[j % 2])
            cur = pltpu.async_copy(rows_v.at[j % 2], x_hbm.at[idx_v.at[j]], osem)
            if prev is not None:
                prev.wait()
            prev = cur
        prev.wait()

    k(rows, idx3, x_ref)


def _layer_tc(q, p_all, s_all, l, Wl, bl, Ws, Tp, D, BT=512):
    nblk = Tp // BT
    off = l * nblk

    def body(q_ref, p_ref, s_ref, w_ref, b_ref, ws_ref, o_ref):
        qb = q_ref[...]
        pb = p_ref[...]
        prob = jax.nn.sigmoid(
            jnp.dot(qb, ws_ref[...], preferred_element_type=jnp.float32))
        z = jnp.dot(qb + pb, w_ref[...],
                    preferred_element_type=jnp.float32) + b_ref[...]
        o_ref[...] = z * prob + pb * s_ref[...]

    return pl.pallas_call(
        body,
        grid=(nblk,),
        in_specs=[
            pl.BlockSpec((BT, D), lambda i: (i, 0)),
            pl.BlockSpec((BT, D), lambda i, off=off: (i + off, 0)),
            pl.BlockSpec((BT, 1), lambda i, off=off: (i + off, 0)),
            pl.BlockSpec((D, D), lambda i: (0, 0)),
            pl.BlockSpec((1, D), lambda i: (0, 0)),
            pl.BlockSpec((D, 1), lambda i: (0, 0)),
        ],
        out_specs=pl.BlockSpec((BT, D), lambda i: (i, 0)),
        out_shape=jax.ShapeDtypeStruct((Tp, D), jnp.float32),
    )(q, p_all, s_all, Wl, bl, Ws)


def _fourier_tc(xy_level, Bf, M, D, BM=1000):
    Dh = D // 2
    xy_cols = [xy_level[:, k:k + 1] for k in range(3)]
    b_rows = [Bf[k:k + 1, :] for k in range(3)]
    two_pi = 2.0 * 3.14159265358979323846

    def body(x0, x1, x2, b0, b1, b2, o_ref):
        def r(v):
            return v[...].astype(jnp.bfloat16).astype(jnp.float32)
        ang = (r(x0) * r(b0) + r(x1) * r(b1) + r(x2) * r(b2)) * two_pi
        o_ref[:, :Dh] = jnp.sin(ang)
        o_ref[:, Dh:] = jnp.cos(ang)

    col = pl.BlockSpec((BM, 1), lambda i: (i, 0))
    row = pl.BlockSpec((1, Dh), lambda i: (0, 0))
    return pl.pallas_call(
        body,
        grid=(M // BM,),
        in_specs=[col, col, col, row, row, row],
        out_specs=pl.BlockSpec((BM, D), lambda i: (i, 0)),
        out_shape=jax.ShapeDtypeStruct((M, D), jnp.float32),
    )(*xy_cols, *b_rows)


def kernel(mem, pos_enc, token_scores, xy_level, W, b, Ws, B_fourier,
           layer_token_indices):
    M, D = mem.shape
    L, T = layer_token_indices.shape
    Tp = -(-T // (NW * C)) * (NW * C)
    K = Tp // (NW * C)

    wrap = jnp.arange(Tp, dtype=jnp.int32) % T
    idx_pad = jnp.take(layer_token_indices.astype(jnp.int32), wrap, axis=1)
    idx3 = idx_pad.reshape(L, NW, K, C)
    idx_lw = idx_pad.reshape(L * NW, K, C)

    Mr = -(-M // 128)
    scores_flat = jnp.concatenate(
        [token_scores, jnp.zeros((Mr * 128 - M,), jnp.float32)])
    scores_mat = scores_flat.reshape(Mr, 128)
    srow_lw = (idx_pad >> 7).reshape(L * NW, K, C)
    scol_lw = (idx_pad & 127).reshape(L * NW, K, C)

    P_all = _p_gather(pos_enc, idx_lw, L, Tp, D)
    S_all = _s_gather(scores_mat, srow_lw, scol_lw, L, Tp).reshape(L * Tp, 1)

    x_ref = jax.new_ref(mem)

    q2_last = None
    for l in range(L):
        q = _q_gather(x_ref, idx3[l], Tp, D)
        q2 = _layer_tc(q, P_all, S_all, l, W[l], b[l].reshape(1, D), Ws,
                       Tp, D)
        if l < L - 1:
            _row_scatter(q2, idx3[l], x_ref, Tp, D)
        else:
            q2_last = q2

    bg = _fourier_tc(xy_level, B_fourier, M, D)
    bg_ref = jax.new_ref(bg)
    _row_scatter(q2_last, idx3[L - 1], bg_ref, Tp, D)
    return jax.freeze(bg_ref)


# speedup vs baseline: 14.4405x; 1.3511x over previous
import functools

import jax
import jax.numpy as jnp
from jax import lax
from jax.experimental import pallas as pl
from jax.experimental.pallas import tpu as pltpu
from jax.experimental.pallas import tpu_sc as plsc

NC = 2
NS = 16
NW = NC * NS
C = 128

_MESH = plsc.VectorSubcoreMesh(core_axis_name="c", subcore_axis_name="s")


def _wid():
    return lax.axis_index("s") * NC + lax.axis_index("c")


def _p_gather(pos_enc, idx_lw, L, Tp, D):
    K = Tp // (NW * C)
    BW = K * C

    @functools.partial(
        pl.kernel,
        out_type=jax.ShapeDtypeStruct((L * Tp, D), jnp.float32),
        mesh=_MESH,
        scratch_types=[
            pltpu.VMEM((K, C), jnp.int32),
            pltpu.VMEM((2, C, D), jnp.float32),
            pltpu.SemaphoreType.DMA,
        ],
    )
    def k(p_hbm, idx_hbm, p_out, idx_v, rows_v, sem):
        w = _wid()
        for l in range(L):
            pltpu.sync_copy(idx_hbm.at[l * NW + w], idx_v)
            base = l * Tp + w * BW
            copies = [None, None]
            copies[0] = pltpu.async_copy(p_hbm.at[idx_v.at[0]], rows_v.at[0], sem)
            for j in range(K):
                nj = j + 1
                if nj < K:
                    copies[nj % 2] = pltpu.async_copy(
                        p_hbm.at[idx_v.at[nj]], rows_v.at[nj % 2], sem)
                copies[j % 2].wait()
                pltpu.sync_copy(rows_v.at[j % 2],
                                p_out.at[pl.ds(base + j * C, C)])

    return k(pos_enc, idx_lw)


def _s_gather(scores_mat, srow_lw, scol_lw, L, Tp):
    K = Tp // (NW * C)
    BW = K * C

    @functools.partial(
        pl.kernel,
        out_type=jax.ShapeDtypeStruct((L * Tp,), jnp.float32),
        mesh=_MESH,
        compiler_params=pltpu.CompilerParams(needs_layout_passes=False),
        scratch_types=[
            pltpu.VMEM((K, C), jnp.int32),
            pltpu.VMEM((K, C), jnp.int32),
            pltpu.VMEM((C, 128), jnp.float32),
            pltpu.VMEM((C,), jnp.float32),
            pltpu.SemaphoreType.DMA,
        ],
    )
    def k(s_hbm, srow_hbm, scol_hbm, s_out, srow_v, scol_v, rows_v, sbuf_v,
          sem):
        w = _wid()
        for l in range(L):
            pltpu.sync_copy(srow_hbm.at[l * NW + w], srow_v)
            pltpu.sync_copy(scol_hbm.at[l * NW + w], scol_v)
            base = l * Tp + w * BW
            for j in range(K):
                pltpu.async_copy(s_hbm.at[srow_v.at[j]], rows_v, sem).wait()
                for g in range(C // 16):
                    rows16 = g * 16 + lax.iota(jnp.int32, 16)
                    cols16 = scol_v[j, pl.ds(g * 16, 16)]
                    sbuf_v[pl.ds(g * 16, 16)] = plsc.load_gather(
                        rows_v, [rows16, cols16])
                pltpu.sync_copy(sbuf_v, s_out.at[pl.ds(base + j * C, C)])

    return k(scores_mat, srow_lw, scol_lw)


def _q_gather(x_ref, idx3, Tp, D):
    K = Tp // (NW * C)
    BW = K * C

    @functools.partial(
        pl.kernel,
        out_type=jax.ShapeDtypeStruct((Tp, D), jnp.float32),
        mesh=_MESH,
        scratch_types=[
            pltpu.VMEM((K, C), jnp.int32),
            pltpu.VMEM((2, C, D), jnp.float32),
            pltpu.SemaphoreType.DMA,
        ],
    )
    def k(x_hbm, idx_hbm, q_out, idx_v, rows_v, sem):
        w = _wid()
        base = w * BW
        pltpu.sync_copy(idx_hbm.at[w], idx_v)
        copies = [None, None]
        copies[0] = pltpu.async_copy(x_hbm.at[idx_v.at[0]], rows_v.at[0], sem)
        for j in range(K):
            nj = j + 1
            if nj < K:
                copies[nj % 2] = pltpu.async_copy(
                    x_hbm.at[idx_v.at[nj]], rows_v.at[nj % 2], sem)
            copies[j % 2].wait()
            pltpu.sync_copy(rows_v.at[j % 2], q_out.at[pl.ds(base + j * C, C)])

    return k(x_ref, idx3)


def _row_scatter(rows, idx3, x_ref, Tp, D):
    K = Tp // (NW * C)
    BW = K * C

    @functools.partial(
        pl.kernel,
        mesh=_MESH,
        scratch_types=[
            pltpu.VMEM((K, C), jnp.int32),
            pltpu.VMEM((2, C, D), jnp.float32),
            pltpu.SemaphoreType.DMA,
            pltpu.SemaphoreType.DMA,
        ],
    )
    def k(rows_hbm, idx_hbm, x_hbm, idx_v, rows_v, isem, osem):
        w = _wid()
        base = w * BW
        pltpu.sync_copy(idx_hbm.at[w], idx_v)
        prev = None
        for j in range(K):
            pltpu.sync_copy(rows_hbm.at[pl.ds(base + j * C, C)], rows_v.at[j % 2])
            cur = pltpu.async_copy(rows_v.at[j % 2], x_hbm.at[idx_v.at[j]], osem)
            if prev is not None:
                prev.wait()
            prev = cur
        prev.wait()

    k(rows, idx3, x_ref)


def _layer_tc(q, p_all, s_all, l, Wl, bl, Ws, Tp, D, BT=512):
    nblk = Tp // BT
    off = l * nblk

    def body(q_ref, p_ref, s_ref, w_ref, b_ref, ws_ref, o_ref):
        qb = q_ref[...]
        pb = p_ref[...]
        prob = jax.nn.sigmoid(
            jnp.dot(qb, ws_ref[...], preferred_element_type=jnp.float32))
        z = jnp.dot(qb + pb, w_ref[...],
                    preferred_element_type=jnp.float32) + b_ref[...]
        sb = s_ref[...].reshape(BT, 1)
        o_ref[...] = z * prob + pb * sb

    return pl.pallas_call(
        body,
        grid=(nblk,),
        in_specs=[
            pl.BlockSpec((BT, D), lambda i: (i, 0)),
            pl.BlockSpec((BT, D), lambda i, off=off: (i + off, 0)),
            pl.BlockSpec((BT,), lambda i, off=off: (i + off,)),
            pl.BlockSpec((D, D), lambda i: (0, 0)),
            pl.BlockSpec((1, D), lambda i: (0, 0)),
            pl.BlockSpec((D, 1), lambda i: (0, 0)),
        ],
        out_specs=pl.BlockSpec((BT, D), lambda i: (i, 0)),
        out_shape=jax.ShapeDtypeStruct((Tp, D), jnp.float32),
    )(q, p_all, s_all, Wl, bl, Ws)


def _fourier_tc(xy_level, Bf, M, D, BM=2000):
    Dh = D // 2

    def body(xy_ref, bf_ref, o_ref):
        xyb = xy_ref[...]

        def r(v):
            return v.astype(jnp.bfloat16).astype(jnp.float32)

        w = (r(xyb[:, 0:1]) * r(bf_ref[0:1, :])
             + r(xyb[:, 1:2]) * r(bf_ref[1:2, :])
             + r(xyb[:, 2:3]) * r(bf_ref[2:3, :]))
        v = w * 4.0
        k = jnp.round(v)
        t = (v - k) * (0.5 * 3.14159265358979323846)
        t2 = t * t
        s = t + t * t2 * (-1.6666654611e-1 + t2 * (8.3321608736e-3
                          + t2 * (-1.9515295891e-4)))
        c = 1.0 + t2 * (-0.5 + t2 * (4.166664568298827e-2
                        + t2 * (-1.388731625493765e-3)))
        m = k.astype(jnp.int32) & 3
        swap = (m & 1) == 1
        sin_base = jnp.where(swap, c, s)
        cos_base = jnp.where(swap, s, c)
        sin_o = jnp.where(m >= 2, -sin_base, sin_base)
        cos_o = jnp.where((m == 1) | (m == 2), -cos_base, cos_base)
        o_ref[:, :Dh] = sin_o
        o_ref[:, Dh:] = cos_o

    return pl.pallas_call(
        body,
        grid=(M // BM,),
        in_specs=[
            pl.BlockSpec((BM, 3), lambda i: (i, 0)),
            pl.BlockSpec((3, Dh), lambda i: (0, 0)),
        ],
        out_specs=pl.BlockSpec((BM, D), lambda i: (i, 0)),
        out_shape=jax.ShapeDtypeStruct((M, D), jnp.float32),
    )(xy_level, Bf)


def kernel(mem, pos_enc, token_scores, xy_level, W, b, Ws, B_fourier,
           layer_token_indices):
    M, D = mem.shape
    L, T = layer_token_indices.shape
    Tp = -(-T // (NW * C)) * (NW * C)
    K = Tp // (NW * C)

    wrap = jnp.arange(Tp, dtype=jnp.int32) % T
    idx_pad = jnp.take(layer_token_indices.astype(jnp.int32), wrap, axis=1)
    idx3 = idx_pad.reshape(L, NW, K, C)
    idx_lw = idx_pad.reshape(L * NW, K, C)

    Mr = -(-M // 128)
    scores_flat = jnp.concatenate(
        [token_scores, jnp.zeros((Mr * 128 - M,), jnp.float32)])
    scores_mat = scores_flat.reshape(Mr, 128)
    srow_lw = (idx_pad >> 7).reshape(L * NW, K, C)
    scol_lw = (idx_pad & 127).reshape(L * NW, K, C)

    x_ref = jax.new_ref(mem)

    P_all = _p_gather(pos_enc, idx_lw, L, Tp, D)
    S_all = _s_gather(scores_mat, srow_lw, scol_lw, L, Tp)

    q2_last = None
    for l in range(L):
        q = _q_gather(x_ref, idx3[l], Tp, D)
        q2 = _layer_tc(q, P_all, S_all, l, W[l], b[l].reshape(1, D), Ws,
                       Tp, D)
        if l < L - 1:
            _row_scatter(q2, idx3[l], x_ref, Tp, D)
        else:
            q2_last = q2

    bg = _fourier_tc(xy_level, B_fourier, M, D)
    bg_ref = jax.new_ref(bg)
    _row_scatter(q2_last, idx3[L - 1], bg_ref, Tp, D)
    return jax.freeze(bg_ref)


# speedup vs baseline: 14.6631x; 1.0154x over previous
import functools

import jax
import jax.numpy as jnp
from jax import lax
from jax.experimental import pallas as pl
from jax.experimental.pallas import tpu as pltpu
from jax.experimental.pallas import tpu_sc as plsc

NC = 2
NS = 16
NW = NC * NS
C = 128

_MESH = plsc.VectorSubcoreMesh(core_axis_name="c", subcore_axis_name="s")


def _wid():
    return lax.axis_index("s") * NC + lax.axis_index("c")


def _ps_gather(pos_enc, scores_mat, idx3, srow3, scol3, Tp, D):
    K = Tp // (NW * C)
    BW = K * C

    @functools.partial(
        pl.kernel,
        out_type=(
            jax.ShapeDtypeStruct((Tp, D), jnp.float32),
            jax.ShapeDtypeStruct((Tp,), jnp.float32),
        ),
        mesh=_MESH,
        compiler_params=pltpu.CompilerParams(needs_layout_passes=False),
        scratch_types=[
            pltpu.VMEM((K, C), jnp.int32),
            pltpu.VMEM((K, C), jnp.int32),
            pltpu.VMEM((K, C), jnp.int32),
            pltpu.VMEM((2, C, D), jnp.float32),
            pltpu.VMEM((C, 128), jnp.float32),
            pltpu.VMEM((C,), jnp.float32),
            pltpu.SemaphoreType.DMA,
            pltpu.SemaphoreType.DMA,
        ],
    )
    def k(p_hbm, s_hbm, idx_hbm, srow_hbm, scol_hbm, p_out, s_out,
          idx_v, srow_v, scol_v, prow_v, srows_v, sbuf_v, psem, ssem):
        w = _wid()
        base = w * BW
        pltpu.sync_copy(idx_hbm.at[w], idx_v)
        pltpu.sync_copy(srow_hbm.at[w], srow_v)
        pltpu.sync_copy(scol_hbm.at[w], scol_v)
        copies = [None, None]
        copies[0] = pltpu.async_copy(p_hbm.at[idx_v.at[0]], prow_v.at[0], psem)
        for j in range(K):
            nj = j + 1
            if nj < K:
                copies[nj % 2] = pltpu.async_copy(
                    p_hbm.at[idx_v.at[nj]], prow_v.at[nj % 2], psem)
            cs = pltpu.async_copy(s_hbm.at[srow_v.at[j]], srows_v, ssem)
            copies[j % 2].wait()
            pltpu.sync_copy(prow_v.at[j % 2],
                            p_out.at[pl.ds(base + j * C, C)])
            cs.wait()
            for g in range(C // 16):
                rows16 = g * 16 + lax.iota(jnp.int32, 16)
                cols16 = scol_v[j, pl.ds(g * 16, 16)]
                sbuf_v[pl.ds(g * 16, 16)] = plsc.load_gather(
                    srows_v, [rows16, cols16])
            pltpu.sync_copy(sbuf_v, s_out.at[pl.ds(base + j * C, C)])

    return k(pos_enc, scores_mat, idx3, srow3, scol3)


def _q_gather(x_table, idx3, Tp, D):
    K = Tp // (NW * C)
    BW = K * C

    @functools.partial(
        pl.kernel,
        out_type=jax.ShapeDtypeStruct((Tp, D), jnp.float32),
        mesh=_MESH,
        scratch_types=[
            pltpu.VMEM((K, C), jnp.int32),
            pltpu.VMEM((2, C, D), jnp.float32),
            pltpu.SemaphoreType.DMA,
        ],
    )
    def k(x_hbm, idx_hbm, q_out, idx_v, rows_v, sem):
        w = _wid()
        base = w * BW
        pltpu.sync_copy(idx_hbm.at[w], idx_v)
        copies = [None, None]
        copies[0] = pltpu.async_copy(x_hbm.at[idx_v.at[0]], rows_v.at[0], sem)
        for j in range(K):
            nj = j + 1
            if nj < K:
                copies[nj % 2] = pltpu.async_copy(
                    x_hbm.at[idx_v.at[nj]], rows_v.at[nj % 2], sem)
            copies[j % 2].wait()
            pltpu.sync_copy(rows_v.at[j % 2], q_out.at[pl.ds(base + j * C, C)])

    return k(x_table, idx3)


def _row_scatter(rows, idx3, x_ref, Tp, D):
    K = Tp // (NW * C)
    BW = K * C

    @functools.partial(
        pl.kernel,
        mesh=_MESH,
        scratch_types=[
            pltpu.VMEM((K, C), jnp.int32),
            pltpu.VMEM((2, C, D), jnp.float32),
            pltpu.SemaphoreType.DMA,
        ],
    )
    def k(rows_hbm, idx_hbm, x_hbm, idx_v, rows_v, osem):
        w = _wid()
        base = w * BW
        pltpu.sync_copy(idx_hbm.at[w], idx_v)
        prev = None
        for j in range(K):
            pltpu.sync_copy(rows_hbm.at[pl.ds(base + j * C, C)],
                            rows_v.at[j % 2])
            cur = pltpu.async_copy(rows_v.at[j % 2], x_hbm.at[idx_v.at[j]],
                                   osem)
            if prev is not None:
                prev.wait()
            prev = cur
        prev.wait()

    k(rows, idx3, x_ref)


def _layer_tc(q, p, s, Wl, bl, Ws, Tp, D, BT=512):
    nblk = Tp // BT

    def body(q_ref, p_ref, s_ref, w_ref, b_ref, ws_ref, o_ref):
        qb = q_ref[...]
        pb = p_ref[...]
        prob = jax.nn.sigmoid(
            jnp.dot(qb, ws_ref[...], preferred_element_type=jnp.float32))
        z = jnp.dot(qb + pb, w_ref[...],
                    preferred_element_type=jnp.float32) + b_ref[...]
        sb = s_ref[...].reshape(BT, 1)
        o_ref[...] = z * prob + pb * sb

    return pl.pallas_call(
        body,
        grid=(nblk,),
        in_specs=[
            pl.BlockSpec((BT, D), lambda i: (i, 0)),
            pl.BlockSpec((BT, D), lambda i: (i, 0)),
            pl.BlockSpec((BT,), lambda i: (i,)),
            pl.BlockSpec((D, D), lambda i: (0, 0)),
            pl.BlockSpec((1, D), lambda i: (0, 0)),
            pl.BlockSpec((D, 1), lambda i: (0, 0)),
        ],
        out_specs=pl.BlockSpec((BT, D), lambda i: (i, 0)),
        out_shape=jax.ShapeDtypeStruct((Tp, D), jnp.float32),
    )(q, p, s, Wl, bl, Ws)


def _fourier_chunk(xy0, xy1, xy2, Bf, M, D, blk_lo, nblk_c, bg_prev, BM=1024):
    Dh = D // 2

    def body(x0, x1, x2, bf_ref, *rest):
        o_ref = rest[-1]

        def r(v):
            return v.astype(jnp.bfloat16).astype(jnp.float32)

        xc0 = r(x0[...].reshape(BM, 1))
        xc1 = r(x1[...].reshape(BM, 1))
        xc2 = r(x2[...].reshape(BM, 1))
        w = (xc0 * r(bf_ref[0:1, :]) + xc1 * r(bf_ref[1:2, :])
             + xc2 * r(bf_ref[2:3, :]))
        v = w * 4.0
        k = jnp.round(v)
        t = (v - k) * (0.5 * 3.14159265358979323846)
        t2 = t * t
        s = t + t * t2 * (-1.6666654611e-1 + t2 * (8.3321608736e-3
                          + t2 * (-1.9515295891e-4)))
        c = 1.0 + t2 * (-0.5 + t2 * (4.166664568298827e-2
                        + t2 * (-1.388731625493765e-3)))
        m = k.astype(jnp.int32) & 3
        swap = (m & 1) == 1
        sin_base = jnp.where(swap, c, s)
        cos_base = jnp.where(swap, s, c)
        sin_o = jnp.where(m >= 2, -sin_base, sin_base)
        cos_o = jnp.where((m == 1) | (m == 2), -cos_base, cos_base)
        o_ref[:, :Dh] = sin_o
        o_ref[:, Dh:] = cos_o

    col = pl.BlockSpec((BM,), lambda i, blk_lo=blk_lo: (i + blk_lo,))
    in_specs = [col, col, col, pl.BlockSpec((3, Dh), lambda i: (0, 0))]
    args = [xy0, xy1, xy2, Bf]
    aliases = {}
    if bg_prev is not None:
        in_specs.append(pl.BlockSpec(memory_space=pl.ANY))
        args.append(bg_prev)
        aliases = {4: 0}
    return pl.pallas_call(
        body,
        grid=(nblk_c,),
        in_specs=in_specs,
        out_specs=pl.BlockSpec((BM, D),
                               lambda i, blk_lo=blk_lo: (i + blk_lo, 0)),
        out_shape=jax.ShapeDtypeStruct((M, D), jnp.float32),
        input_output_aliases=aliases,
    )(*args)


def kernel(mem, pos_enc, token_scores, xy_level, W, b, Ws, B_fourier,
           layer_token_indices):
    M, D = mem.shape
    L, T = layer_token_indices.shape
    Tp = -(-T // (NW * C)) * (NW * C)
    K = Tp // (NW * C)

    wrap = jnp.arange(Tp, dtype=jnp.int32) % T
    idx_pad = jnp.take(layer_token_indices.astype(jnp.int32), wrap, axis=1)
    idx3 = idx_pad.reshape(L, NW, K, C)

    Mr = -(-M // 128)
    scores_flat = jnp.concatenate(
        [token_scores, jnp.zeros((Mr * 128 - M,), jnp.float32)])
    scores_mat = scores_flat.reshape(Mr, 128)
    srow3 = (idx_pad >> 7).reshape(L, NW, K, C)
    scol3 = (idx_pad & 127).reshape(L, NW, K, C)

    xy0 = xy_level[:, 0]
    xy1 = xy_level[:, 1]
    xy2 = xy_level[:, 2]

    BM = 1024
    nblk_total = -(-M // BM)
    base_c = nblk_total // L
    nblks = [base_c + (1 if i < nblk_total - base_c * L else 0)
             for i in range(L)]
    blk_lo = [sum(nblks[:i]) for i in range(L)]

    x_ref = jax.new_ref(mem)

    def ps(l):
        return _ps_gather(pos_enc, scores_mat, idx3[l], srow3[l], scol3[l],
                          Tp, D)

    P0, S0 = ps(0)
    q = _q_gather(mem, idx3[0], Tp, D)
    P1, S1 = ps(1)
    Pc, Sc = P0, S0
    Pn, Sn = P1, S1
    bg = None
    q2 = None
    for l in range(L):
        q2 = _layer_tc(q, Pc, Sc, W[l], b[l].reshape(1, D), Ws, Tp, D)
        bg = _fourier_chunk(xy0, xy1, xy2, B_fourier, M, D,
                            blk_lo[l], nblks[l], bg, BM=BM)
        if l < L - 1:
            _row_scatter(q2, idx3[l], x_ref, Tp, D)
            q = _q_gather(x_ref, idx3[l + 1], Tp, D)
            Pc, Sc = Pn, Sn
            if l + 2 < L:
                Pn, Sn = ps(l + 2)

    bg_ref = jax.new_ref(bg)
    _row_scatter(q2, idx3[L - 1], bg_ref, Tp, D)
    return jax.freeze(bg_ref)


# speedup vs baseline: 16.4120x; 1.1193x over previous
import functools

import jax
import jax.numpy as jnp
from jax import lax
from jax.experimental import pallas as pl
from jax.experimental.pallas import tpu as pltpu
from jax.experimental.pallas import tpu_sc as plsc

NC = 2
NS = 16
NW = NC * NS
C = 128

_MESH = plsc.VectorSubcoreMesh(core_axis_name="c", subcore_axis_name="s")


def _wid():
    return lax.axis_index("s") * NC + lax.axis_index("c")


def _ps_gather(pos_enc, scores_mat, idx3, srow3, scol3, Tp, D):
    K = Tp // (NW * C)
    BW = K * C

    @functools.partial(
        pl.kernel,
        out_type=(
            jax.ShapeDtypeStruct((Tp, D), jnp.float32),
            jax.ShapeDtypeStruct((Tp,), jnp.float32),
        ),
        mesh=_MESH,
        compiler_params=pltpu.CompilerParams(needs_layout_passes=False),
        scratch_types=[
            pltpu.VMEM((K, C), jnp.int32),
            pltpu.VMEM((K, C), jnp.int32),
            pltpu.VMEM((K, C), jnp.int32),
            pltpu.VMEM((2, C, D), jnp.float32),
            pltpu.VMEM((C, 128), jnp.float32),
            pltpu.VMEM((C,), jnp.float32),
            pltpu.SemaphoreType.DMA,
            pltpu.SemaphoreType.DMA,
        ],
    )
    def k(p_hbm, s_hbm, idx_hbm, srow_hbm, scol_hbm, p_out, s_out,
          idx_v, srow_v, scol_v, prow_v, srows_v, sbuf_v, psem, ssem):
        w = _wid()
        base = w * BW
        pltpu.sync_copy(idx_hbm.at[w], idx_v)
        pltpu.sync_copy(srow_hbm.at[w], srow_v)
        pltpu.sync_copy(scol_hbm.at[w], scol_v)
        copies = [None, None]
        copies[0] = pltpu.async_copy(p_hbm.at[idx_v.at[0]], prow_v.at[0], psem)
        for j in range(K):
            nj = j + 1
            if nj < K:
                copies[nj % 2] = pltpu.async_copy(
                    p_hbm.at[idx_v.at[nj]], prow_v.at[nj % 2], psem)
            cs = pltpu.async_copy(s_hbm.at[srow_v.at[j]], srows_v, ssem)
            copies[j % 2].wait()
            pltpu.sync_copy(prow_v.at[j % 2],
                            p_out.at[pl.ds(base + j * C, C)])
            cs.wait()
            for g in range(C // 16):
                rows16 = g * 16 + lax.iota(jnp.int32, 16)
                cols16 = scol_v[j, pl.ds(g * 16, 16)]
                sbuf_v[pl.ds(g * 16, 16)] = plsc.load_gather(
                    srows_v, [rows16, cols16])
            pltpu.sync_copy(sbuf_v, s_out.at[pl.ds(base + j * C, C)])

    return k(pos_enc, scores_mat, idx3, srow3, scol3)


def _q_gather(x_table, idx3, Tp, D):
    K = Tp // (NW * C)
    BW = K * C

    @functools.partial(
        pl.kernel,
        out_type=jax.ShapeDtypeStruct((Tp, D), jnp.float32),
        mesh=_MESH,
        scratch_types=[
            pltpu.VMEM((K, C), jnp.int32),
            pltpu.VMEM((2, C, D), jnp.float32),
            pltpu.SemaphoreType.DMA,
        ],
    )
    def k(x_hbm, idx_hbm, q_out, idx_v, rows_v, sem):
        w = _wid()
        base = w * BW
        pltpu.sync_copy(idx_hbm.at[w], idx_v)
        copies = [None, None]
        copies[0] = pltpu.async_copy(x_hbm.at[idx_v.at[0]], rows_v.at[0], sem)
        for j in range(K):
            nj = j + 1
            if nj < K:
                copies[nj % 2] = pltpu.async_copy(
                    x_hbm.at[idx_v.at[nj]], rows_v.at[nj % 2], sem)
            copies[j % 2].wait()
            pltpu.sync_copy(rows_v.at[j % 2], q_out.at[pl.ds(base + j * C, C)])

    return k(x_table, idx3)


def _row_scatter(rows, idx3, x_ref, Tp, D):
    K = Tp // (NW * C)
    BW = K * C

    @functools.partial(
        pl.kernel,
        mesh=_MESH,
        scratch_types=[
            pltpu.VMEM((K, C), jnp.int32),
            pltpu.VMEM((2, C, D), jnp.float32),
            pltpu.SemaphoreType.DMA,
        ],
    )
    def k(rows_hbm, idx_hbm, x_hbm, idx_v, rows_v, osem):
        w = _wid()
        base = w * BW
        pltpu.sync_copy(idx_hbm.at[w], idx_v)
        prev = None
        for j in range(K):
            pltpu.sync_copy(rows_hbm.at[pl.ds(base + j * C, C)],
                            rows_v.at[j % 2])
            cur = pltpu.async_copy(rows_v.at[j % 2], x_hbm.at[idx_v.at[j]],
                                   osem)
            if prev is not None:
                prev.wait()
            prev = cur
        prev.wait()

    k(rows, idx3, x_ref)


def _layer_tc(q, p, s, Wl, bl, Ws, Tp, D, BT=512):
    nblk = Tp // BT

    def body(q_ref, p_ref, s_ref, w_ref, b_ref, ws_ref, o_ref):
        qb = q_ref[...]
        pb = p_ref[...]
        prob = jax.nn.sigmoid(
            jnp.dot(qb, ws_ref[...], preferred_element_type=jnp.float32))
        z = jnp.dot(qb + pb, w_ref[...],
                    preferred_element_type=jnp.float32) + b_ref[...]
        sb = s_ref[...].reshape(BT, 1)
        o_ref[...] = z * prob + pb * sb

    return pl.pallas_call(
        body,
        grid=(nblk,),
        in_specs=[
            pl.BlockSpec((BT, D), lambda i: (i, 0)),
            pl.BlockSpec((BT, D), lambda i: (i, 0)),
            pl.BlockSpec((BT,), lambda i: (i,)),
            pl.BlockSpec((D, D), lambda i: (0, 0)),
            pl.BlockSpec((1, D), lambda i: (0, 0)),
            pl.BlockSpec((D, 1), lambda i: (0, 0)),
        ],
        out_specs=pl.BlockSpec((BT, D), lambda i: (i, 0)),
        out_shape=jax.ShapeDtypeStruct((Tp, D), jnp.float32),
    )(q, p, s, Wl, bl, Ws)


def _fourier_chunk(xy0, xy1, xy2, Bf, M, D, blk_lo, nblk_c, bg_prev, BM=1024):
    Dh = D // 2

    def body(x0, x1, x2, bf_ref, *rest):
        o_ref = rest[-1]

        def r(v):
            return v.astype(jnp.bfloat16).astype(jnp.float32)

        xc0 = r(x0[...].reshape(BM, 1))
        xc1 = r(x1[...].reshape(BM, 1))
        xc2 = r(x2[...].reshape(BM, 1))
        w = (xc0 * r(bf_ref[0:1, :]) + xc1 * r(bf_ref[1:2, :])
             + xc2 * r(bf_ref[2:3, :]))
        v = w * 4.0
        k = jnp.round(v)
        t = (v - k) * (0.5 * 3.14159265358979323846)
        t2 = t * t
        s = t + t * t2 * (-1.6666654611e-1 + t2 * (8.3321608736e-3
                          + t2 * (-1.9515295891e-4)))
        c = 1.0 + t2 * (-0.5 + t2 * (4.166664568298827e-2
                        + t2 * (-1.388731625493765e-3)))
        m = k.astype(jnp.int32) & 3
        swap = (m & 1) == 1
        sin_base = jnp.where(swap, c, s)
        cos_base = jnp.where(swap, s, c)
        sin_o = jnp.where(m >= 2, -sin_base, sin_base)
        cos_o = jnp.where((m == 1) | (m == 2), -cos_base, cos_base)
        o_ref[:, :Dh] = sin_o
        o_ref[:, Dh:] = cos_o

    col = pl.BlockSpec((BM,), lambda i, blk_lo=blk_lo: (i + blk_lo,))
    in_specs = [col, col, col, pl.BlockSpec((3, Dh), lambda i: (0, 0))]
    args = [xy0, xy1, xy2, Bf]
    aliases = {}
    if bg_prev is not None:
        in_specs.append(pl.BlockSpec(memory_space=pl.ANY))
        args.append(bg_prev)
        aliases = {4: 0}
    return pl.pallas_call(
        body,
        grid=(nblk_c,),
        in_specs=in_specs,
        out_specs=pl.BlockSpec((BM, D),
                               lambda i, blk_lo=blk_lo: (i + blk_lo, 0)),
        out_shape=jax.ShapeDtypeStruct((M, D), jnp.float32),
        input_output_aliases=aliases,
    )(*args)


def kernel(mem, pos_enc, token_scores, xy_level, W, b, Ws, B_fourier,
           layer_token_indices):
    M, D = mem.shape
    L, T = layer_token_indices.shape
    Tp = -(-T // (NW * C)) * (NW * C)
    K = Tp // (NW * C)

    wrap = jnp.arange(Tp, dtype=jnp.int32) % T
    idx_pad = jnp.take(layer_token_indices.astype(jnp.int32), wrap, axis=1)
    idx3 = idx_pad.reshape(L, NW, K, C)

    Mr = -(-M // 128)
    scores_flat = jnp.concatenate(
        [token_scores, jnp.zeros((Mr * 128 - M,), jnp.float32)])
    scores_mat = scores_flat.reshape(Mr, 128)
    srow3 = (idx_pad >> 7).reshape(L, NW, K, C)
    scol3 = (idx_pad & 127).reshape(L, NW, K, C)

    xy0 = xy_level[:, 0]
    xy1 = xy_level[:, 1]
    xy2 = xy_level[:, 2]

    BM = 1024
    nblk_total = -(-M // BM)
    base_c = nblk_total // L
    nblks = [base_c + (1 if i < nblk_total - base_c * L else 0)
             for i in range(L)]
    blk_lo = [sum(nblks[:i]) for i in range(L)]

    x_ref = jax.new_ref(mem)

    def ps(l):
        return _ps_gather(pos_enc, scores_mat, idx3[l], srow3[l], scol3[l],
                          Tp, D)

    P0, S0 = ps(0)
    q = _q_gather(mem, idx3[0], Tp, D)
    P1, S1 = ps(1)
    Pc, Sc = P0, S0
    Pn, Sn = P1, S1
    bg = None
    q2 = None
    for l in range(L):
        q2 = _layer_tc(q, Pc, Sc, W[l], b[l].reshape(1, D), Ws, Tp, D)
        if bg is None:
            g0, g1, g2, _ = lax.optimization_barrier((xy0, xy1, xy2, q2))
            bgp = None
        else:
            g0, g1, g2, bgp, _ = lax.optimization_barrier(
                (xy0, xy1, xy2, bg, q2))
        bg = _fourier_chunk(g0, g1, g2, B_fourier, M, D,
                            blk_lo[l], nblks[l], bgp, BM=BM)
        if l < L - 1:
            _row_scatter(q2, idx3[l], x_ref, Tp, D)
            q = _q_gather(x_ref, idx3[l + 1], Tp, D)
            q, bg = lax.optimization_barrier((q, bg))
            Pc, Sc = Pn, Sn
            if l + 2 < L:
                Pn, Sn = ps(l + 2)

    bg_ref = jax.new_ref(bg)
    _row_scatter(q2, idx3[L - 1], bg_ref, Tp, D)
    return jax.freeze(bg_ref)
